# baseline (speedup 1.0000x reference)
import jax
import jax.numpy as jnp
from jax.experimental import pallas as pl
from jax.experimental.pallas import tpu as pltpu

N = 4096
TM = 512


def _body(prev_ref, nblk_ref, nfull_ref, a_ref, wphi_ref, wth_ref, out_ref,
          gs_ref):
    i = pl.program_id(0)
    w0 = wth_ref[0, 0]
    w1 = wth_ref[1, 0]
    w2 = wth_ref[2, 0]
    c0 = w0 * w0
    c1 = w1 * w1
    c2 = w2 * w2

    @pl.when(i == 0)
    def _init():
        xT = nfull_ref[:, :].T
        x0 = xT[0:1, :]
        x1 = xT[1:2, :]
        x2 = xT[2:3, :]
        gs_ref[0:1, :] = x0 * (-2.0 * c0)
        gs_ref[1:2, :] = x1 * (-2.0 * c1)
        gs_ref[2:3, :] = x2 * (-2.0 * c2)
        gs_ref[3:4, :] = x0 * x0 * c0 + x1 * x1 * c1 + x2 * x2 * c2

    g0 = gs_ref[0:1, :]
    g1 = gs_ref[1:2, :]
    g2 = gs_ref[2:3, :]
    sqj = gs_ref[3:4, :]

    y0 = nblk_ref[:, 0:1]
    y1 = nblk_ref[:, 1:2]
    y2 = nblk_ref[:, 2:3]
    sqi = y0 * y0 * c0 + y1 * y1 * c1 + y2 * y2 * c2

    z = ((sqj + y0 * g0) + y1 * g1) + y2 * g2

    mask = a_ref[:, :] != 0
    neg = jnp.float32(-jnp.inf)
    m = jnp.max(jnp.where(mask, z, neg), axis=1, keepdims=True)
    d2 = sqi + m
    maxd = jnp.where(m == neg, neg, jnp.sqrt(jnp.maximum(d2, 0.0)))

    wmean = jnp.mean(wphi_ref[0, :])
    out_ref[:, :] = 0.5 * (prev_ref[:, :] + maxd * wmean)


@jax.jit
def _run(prev, nodes, adjacency, wphi, wth):
    prev = prev.reshape(N, 1)
    wphi = wphi.reshape(1, -1)
    grid = (N // TM,)
    out = pl.pallas_call(
        _body,
        grid=grid,
        in_specs=[
            pl.BlockSpec((TM, 1), lambda i: (i, 0)),
            pl.BlockSpec((TM, 3), lambda i: (i, 0)),
            pl.BlockSpec((N, 3), lambda i: (0, 0)),
            pl.BlockSpec((TM, N), lambda i: (i, 0)),
            pl.BlockSpec((1, wphi.shape[1]), lambda i: (0, 0)),
            pl.BlockSpec((3, 1), lambda i: (0, 0)),
        ],
        out_specs=pl.BlockSpec((TM, 1), lambda i: (i, 0)),
        out_shape=jax.ShapeDtypeStruct((N, 1), jnp.float32),
        scratch_shapes=[pltpu.VMEM((4, N), jnp.float32)],
    )(prev, nodes, nodes, adjacency, wphi, wth)
    return out.reshape(N)


def kernel(previous_inclusion_score, nodes, adjacency_matrix, W_phi, W_theta):
    return _run(previous_inclusion_score, nodes, adjacency_matrix, W_phi,
                W_theta)


# speedup vs baseline: 1.3131x; 1.3131x over previous
import jax
import jax.numpy as jnp
from jax.experimental import pallas as pl

N = 4096
TM = 512


def _body(prev_ref, nblk_ref, ntT_ref, a_ref, wphi_ref, wth_ref, out_ref):
    i = pl.program_id(0)
    w0 = wth_ref[0, 0]
    w1 = wth_ref[1, 0]
    w2 = wth_ref[2, 0]
    c0 = w0 * w0
    c1 = w1 * w1
    c2 = w2 * w2

    x0 = ntT_ref[0:1, :]
    x1 = ntT_ref[1:2, :]
    x2 = ntT_ref[2:3, :]
    g0 = x0 * (-2.0 * c0)
    g1 = x1 * (-2.0 * c1)
    g2 = x2 * (-2.0 * c2)
    sq = x0 * x0 * c0 + x1 * x1 * c1 + x2 * x2 * c2

    y0 = nblk_ref[:, 0:1]
    y1 = nblk_ref[:, 1:2]
    y2 = nblk_ref[:, 2:3]

    z = ((sq + y0 * g0) + y1 * g1) + y2 * g2

    mask = a_ref[:, :] != 0
    neg = jnp.float32(-jnp.inf)
    m = jnp.max(jnp.where(mask, z, neg), axis=1, keepdims=True)
    mrow = m.T
    xi0 = ntT_ref[0:1, pl.ds(i * TM, TM)]
    xi1 = ntT_ref[1:2, pl.ds(i * TM, TM)]
    xi2 = ntT_ref[2:3, pl.ds(i * TM, TM)]
    sqi = xi0 * xi0 * c0 + xi1 * xi1 * c1 + xi2 * xi2 * c2
    d2 = sqi + mrow
    maxd = jnp.where(mrow == neg, neg, jnp.sqrt(jnp.maximum(d2, 0.0)))

    half_wmean = 0.5 * jnp.mean(wphi_ref[0, :])
    out_ref[0:1, :] = 0.5 * prev_ref[0:1, :] + maxd * half_wmean


@jax.jit
def _run(prev, nodes, adjacency, wphi, wth):
    prev = prev.reshape(1, N)
    wphi = wphi.reshape(1, -1)
    ntT = nodes.T
    grid = (N // TM,)
    out = pl.pallas_call(
        _body,
        grid=grid,
        in_specs=[
            pl.BlockSpec((1, TM), lambda i: (0, i)),
            pl.BlockSpec((TM, 3), lambda i: (i, 0)),
            pl.BlockSpec((3, N), lambda i: (0, 0)),
            pl.BlockSpec((TM, N), lambda i: (i, 0)),
            pl.BlockSpec((1, wphi.shape[1]), lambda i: (0, 0)),
            pl.BlockSpec((3, 1), lambda i: (0, 0)),
        ],
        out_specs=pl.BlockSpec((1, TM), lambda i: (0, i)),
        out_shape=jax.ShapeDtypeStruct((1, N), jnp.float32),
    )(prev, nodes, ntT, adjacency, wphi, wth)
    return out.reshape(N)


def kernel(previous_inclusion_score, nodes, adjacency_matrix, W_phi, W_theta):
    return _run(previous_inclusion_score, nodes, adjacency_matrix, W_phi,
                W_theta)
